# baseline (speedup 1.0000x reference)
import functools

import jax
import jax.numpy as jnp
from jax.experimental import pallas as pl
from jax.experimental.pallas import tpu as pltpu

N, E = 1024, 2048
NFEAT_V, NFEAT_E, NHID, NCLASS = 128, 16, 64, 16
BN = 256
BE = 256


def _node_kernel(T_blk, T_full, adj_blk, Hv, He, W, p, b,
                 out_ref, d_scr, HW_scr, *, log_sm):
    i = pl.program_id(0)

    @pl.when(i == 0)
    def _init():
        heb = He[...].astype(jnp.bfloat16).astype(jnp.float32)
        pb = p[...].astype(jnp.bfloat16).astype(jnp.float32)
        d_scr[...] = jnp.sum(heb * pb, axis=1).reshape(1, E)
        HW_scr[...] = jnp.dot(Hv[...], W[...],
                              preferred_element_type=jnp.float32)

    mult = jax.lax.dot_general(
        T_blk[...] * d_scr[...], T_full[...],
        (((1,), (1,)), ((), ())), preferred_element_type=jnp.float32)
    rows = i * BN + jax.lax.broadcasted_iota(jnp.int32, (BN, N), 0)
    cols = jax.lax.broadcasted_iota(jnp.int32, (BN, N), 1)
    A = jnp.where(rows == cols, 1.0, mult) * adj_blk[...]
    out = jnp.dot(A, HW_scr[...], preferred_element_type=jnp.float32) + b[...]
    if log_sm:
        shifted = out - jnp.max(out, axis=1, keepdims=True)
        out = shifted - jnp.log(jnp.sum(jnp.exp(shifted), axis=1,
                                        keepdims=True))
    else:
        out = jnp.maximum(out, 0.0)
    out_ref[...] = out


def _edge_kernel(T_cols, T_full, adj_blk, Hv, He, W, p, b,
                 out_ref, d_scr, HW_scr):
    j = pl.program_id(0)

    @pl.when(j == 0)
    def _init():
        hvb = Hv[...].astype(jnp.bfloat16).astype(jnp.float32)
        pb = p[...].astype(jnp.bfloat16).astype(jnp.float32)
        d_scr[...] = jnp.sum(hvb * pb, axis=1, keepdims=True)
        HW_scr[...] = jnp.dot(jnp.maximum(He[...], 0.0), W[...],
                              preferred_element_type=jnp.float32)

    mult = jax.lax.dot_general(
        T_cols[...] * d_scr[...], T_full[...],
        (((0,), (0,)), ((), ())), preferred_element_type=jnp.float32)
    rows = j * BE + jax.lax.broadcasted_iota(jnp.int32, (BE, E), 0)
    cols = jax.lax.broadcasted_iota(jnp.int32, (BE, E), 1)
    A = jnp.where(rows == cols, 1.0, mult) * adj_blk[...]
    out = jnp.dot(A, HW_scr[...], preferred_element_type=jnp.float32) + b[...]
    out_ref[...] = jnp.maximum(out, 0.0)


def _node_call(T, adj_v, Hv, He, W, p, b, nin, nout, log_sm):
    return pl.pallas_call(
        functools.partial(_node_kernel, log_sm=log_sm),
        grid=(N // BN,),
        in_specs=[
            pl.BlockSpec((BN, E), lambda i: (i, 0)),
            pl.BlockSpec((N, E), lambda i: (0, 0)),
            pl.BlockSpec((BN, N), lambda i: (i, 0)),
            pl.BlockSpec((N, nin), lambda i: (0, 0)),
            pl.BlockSpec((E, NFEAT_E), lambda i: (0, 0)),
            pl.BlockSpec((nin, nout), lambda i: (0, 0)),
            pl.BlockSpec((1, NFEAT_E), lambda i: (0, 0)),
            pl.BlockSpec((1, nout), lambda i: (0, 0)),
        ],
        out_specs=pl.BlockSpec((BN, nout), lambda i: (i, 0)),
        out_shape=jax.ShapeDtypeStruct((N, nout), jnp.float32),
        scratch_shapes=[pltpu.VMEM((1, E), jnp.float32),
                        pltpu.VMEM((N, nout), jnp.float32)],
    )(T, T, adj_v, Hv, He, W, p, b)


def _edge_call(T, adj_e, Hv, He, W, p, b):
    return pl.pallas_call(
        _edge_kernel,
        grid=(E // BE,),
        in_specs=[
            pl.BlockSpec((N, BE), lambda j: (0, j)),
            pl.BlockSpec((N, E), lambda j: (0, 0)),
            pl.BlockSpec((BE, E), lambda j: (j, 0)),
            pl.BlockSpec((N, NHID), lambda j: (0, 0)),
            pl.BlockSpec((E, NFEAT_E), lambda j: (0, 0)),
            pl.BlockSpec((NFEAT_E, NFEAT_E), lambda j: (0, 0)),
            pl.BlockSpec((1, NHID), lambda j: (0, 0)),
            pl.BlockSpec((1, NFEAT_E), lambda j: (0, 0)),
        ],
        out_specs=pl.BlockSpec((BE, NFEAT_E), lambda j: (j, 0)),
        out_shape=jax.ShapeDtypeStruct((E, NFEAT_E), jnp.float32),
        scratch_shapes=[pltpu.VMEM((N, 1), jnp.float32),
                        pltpu.VMEM((E, NFEAT_E), jnp.float32)],
    )(T, T, adj_e, Hv, He, W, p, b)


def kernel(X, Z, adj_e, adj_v, T, W1, p1, b1, W2, p2, b2, W3, p3, b3):
    b1r, b2r, b3r = b1.reshape(1, -1), b2.reshape(1, -1), b3.reshape(1, -1)
    Xh = _node_call(T, adj_v, X, Z, W1, p1, b1r, NFEAT_V, NHID, log_sm=False)
    Zh = _edge_call(T, adj_e, Xh, Z, W2, p2, b2r)
    return _node_call(T, adj_v, Xh, Zh, W3, p3, b3r, NHID, NCLASS, log_sm=True)


# speedup vs baseline: 1.2253x; 1.2253x over previous
import jax
import jax.numpy as jnp
from jax.experimental import pallas as pl
from jax.experimental.pallas import tpu as pltpu

N, E = 1024, 2048
NFEAT_V, NFEAT_E, NHID, NCLASS = 128, 16, 64, 16
BN = 256
BE = 256
PH1, PH2 = 4, 12


def _bf(x):
    return x.astype(jnp.bfloat16).astype(jnp.float32)


def _fused_kernel(T_ref, Tc_ref, adj_v_ref, adj_e_ref, X_ref, Z_ref,
                  W1_ref, p1_ref, b1_ref, W2_ref, p2_ref, b2_ref,
                  W3_ref, p3_ref, b3_ref, out_ref,
                  Xh, Zh, HW, ZW, d1, d2, d3):
    s = pl.program_id(0)

    @pl.when(s == 0)
    def _init1():
        d1[...] = jnp.sum(_bf(Z_ref[...]) * _bf(p1_ref[...]),
                          axis=1).reshape(1, E)
        HW[:, :NHID] = jnp.dot(X_ref[...], W1_ref[...],
                               preferred_element_type=jnp.float32)

    @pl.when(s < PH1)
    def _gc1():
        i = s
        T_blk = T_ref[pl.ds(i * BN, BN), :]
        mult = jax.lax.dot_general(
            T_blk * d1[...], T_ref[...], (((1,), (1,)), ((), ())),
            preferred_element_type=jnp.float32)
        rows = i * BN + jax.lax.broadcasted_iota(jnp.int32, (BN, N), 0)
        cols = jax.lax.broadcasted_iota(jnp.int32, (BN, N), 1)
        A = jnp.where(rows == cols, 1.0, mult) * adj_v_ref[pl.ds(i * BN, BN), :]
        out = jnp.dot(A, HW[:, :NHID],
                      preferred_element_type=jnp.float32) + b1_ref[...]
        Xh[pl.ds(i * BN, BN), :] = jnp.maximum(out, 0.0)

    @pl.when(s == PH1)
    def _init2():
        d2[...] = jnp.sum(_bf(Xh[...]) * _bf(p2_ref[...]), axis=1,
                          keepdims=True)
        ZW[...] = jnp.dot(jnp.maximum(Z_ref[...], 0.0), W2_ref[...],
                          preferred_element_type=jnp.float32)

    @pl.when((s >= PH1) & (s < PH2))
    def _gc2():
        j = s - PH1
        mult = jax.lax.dot_general(
            Tc_ref[...] * d2[...], T_ref[...], (((0,), (0,)), ((), ())),
            preferred_element_type=jnp.float32)
        rows = j * BE + jax.lax.broadcasted_iota(jnp.int32, (BE, E), 0)
        cols = jax.lax.broadcasted_iota(jnp.int32, (BE, E), 1)
        A = jnp.where(rows == cols, 1.0, mult) * adj_e_ref[...]
        out = jnp.dot(A, ZW[...],
                      preferred_element_type=jnp.float32) + b2_ref[...]
        Zh[pl.ds(j * BE, BE), :] = jnp.maximum(out, 0.0)

    @pl.when(s == PH2)
    def _init3():
        d3[...] = jnp.sum(_bf(Zh[...]) * _bf(p3_ref[...]),
                          axis=1).reshape(1, E)
        HW[:, :NCLASS] = jnp.dot(Xh[...], W3_ref[...],
                                 preferred_element_type=jnp.float32)

    @pl.when(s >= PH2)
    def _gc3():
        i = s - PH2
        T_blk = T_ref[pl.ds(i * BN, BN), :]
        mult = jax.lax.dot_general(
            T_blk * d3[...], T_ref[...], (((1,), (1,)), ((), ())),
            preferred_element_type=jnp.float32)
        rows = i * BN + jax.lax.broadcasted_iota(jnp.int32, (BN, N), 0)
        cols = jax.lax.broadcasted_iota(jnp.int32, (BN, N), 1)
        A = jnp.where(rows == cols, 1.0, mult) * adj_v_ref[pl.ds(i * BN, BN), :]
        out = jnp.dot(A, HW[:, :NCLASS],
                      preferred_element_type=jnp.float32) + b3_ref[...]
        shifted = out - jnp.max(out, axis=1, keepdims=True)
        out_ref[...] = shifted - jnp.log(jnp.sum(jnp.exp(shifted), axis=1,
                                                 keepdims=True))


def kernel(X, Z, adj_e, adj_v, T, W1, p1, b1, W2, p2, b2, W3, p3, b3):
    b1r, b2r, b3r = b1.reshape(1, -1), b2.reshape(1, -1), b3.reshape(1, -1)
    const = lambda a, b: (lambda s: (a, b))
    return pl.pallas_call(
        _fused_kernel,
        grid=(16,),
        in_specs=[
            pl.BlockSpec((N, E), const(0, 0)),
            pl.BlockSpec((N, BE), lambda s: (0, jnp.clip(s - PH1, 0, 7))),
            pl.BlockSpec((N, N), const(0, 0)),
            pl.BlockSpec((BE, E), lambda s: (jnp.clip(s - PH1, 0, 7), 0)),
            pl.BlockSpec((N, NFEAT_V), const(0, 0)),
            pl.BlockSpec((E, NFEAT_E), const(0, 0)),
            pl.BlockSpec((NFEAT_V, NHID), const(0, 0)),
            pl.BlockSpec((1, NFEAT_E), const(0, 0)),
            pl.BlockSpec((1, NHID), const(0, 0)),
            pl.BlockSpec((NFEAT_E, NFEAT_E), const(0, 0)),
            pl.BlockSpec((1, NHID), const(0, 0)),
            pl.BlockSpec((1, NFEAT_E), const(0, 0)),
            pl.BlockSpec((NHID, NCLASS), const(0, 0)),
            pl.BlockSpec((1, NFEAT_E), const(0, 0)),
            pl.BlockSpec((1, NCLASS), const(0, 0)),
        ],
        out_specs=pl.BlockSpec((BN, NCLASS),
                               lambda s: (jnp.clip(s - PH2, 0, 3), 0)),
        out_shape=jax.ShapeDtypeStruct((N, NCLASS), jnp.float32),
        scratch_shapes=[
            pltpu.VMEM((N, NHID), jnp.float32),
            pltpu.VMEM((E, NFEAT_E), jnp.float32),
            pltpu.VMEM((N, NHID), jnp.float32),
            pltpu.VMEM((E, NFEAT_E), jnp.float32),
            pltpu.VMEM((1, E), jnp.float32),
            pltpu.VMEM((N, 1), jnp.float32),
            pltpu.VMEM((1, E), jnp.float32),
        ],
    )(T, T, adj_v, adj_e, X, Z, W1, p1, b1r, W2, p2, b2r, W3, p3, b3r)
